# Initial kernel scaffold; baseline (speedup 1.0000x reference)
import jax
import jax.numpy as jnp
from jax.experimental import pallas as pl


def kernel(input, adj_edge_index, adj_edge_weight, W, b):
    raise NotImplementedError("write your pallas kernel here")


# speedup vs baseline: 3.9128x; 3.9128x over previous
import functools

import jax
import jax.numpy as jnp
from jax import lax
from jax.experimental import pallas as pl
from jax.experimental.pallas import tpu as pltpu
from jax.experimental.pallas import tpu_sc as plsc

N = 10000
D = 128
E = 320000

NC = 2
NS = 16
NW = NC * NS
CHUNK = 128
CH = -(-E // (NW * CHUNK))
E_PAD = NW * CH * CHUNK

ROWS_PER_TILE = 632
ROWS_LAST = N - (NS - 1) * ROWS_PER_TILE


def _mm_body(x_ref, w_ref, b_ref, o_ref):
    o_ref[...] = lax.dot_general(
        x_ref[...], w_ref[...], (((1,), (1,)), ((), ())),
        preferred_element_type=jnp.float32,
    ) + b_ref[...]


def _matmul(x, W, b):
    return pl.pallas_call(
        _mm_body,
        grid=(10,),
        in_specs=[
            pl.BlockSpec((N // 10, D), lambda i: (i, 0)),
            pl.BlockSpec((D, D), lambda i: (0, 0)),
            pl.BlockSpec((1, D), lambda i: (0, 0)),
        ],
        out_specs=pl.BlockSpec((N // 10, D), lambda i: (i, 0)),
        out_shape=jax.ShapeDtypeStruct((N, D), jnp.float32),
    )(x, W, b.reshape(1, D))


def _add_body(p_ref, o_ref):
    o_ref[...] = p_ref[0] + p_ref[1]


def _sum_partials(partials):
    return pl.pallas_call(
        _add_body,
        grid=(10,),
        in_specs=[pl.BlockSpec((2, N // 10, D), lambda i: (0, i, 0))],
        out_specs=pl.BlockSpec((N // 10, D), lambda i: (i, 0)),
        out_shape=jax.ShapeDtypeStruct((N, D), jnp.float32),
    )(partials)


@functools.cache
def _build_sc_scatter():
    mesh = plsc.VectorSubcoreMesh(
        core_axis_name="c", subcore_axis_name="s", num_cores=NC, num_subcores=NS
    )
    return pl.kernel(
        _sc_scatter_body,
        out_type=jax.ShapeDtypeStruct((NC, N, D), jnp.float32),
        mesh=mesh,
        scratch_types=[
            pltpu.VMEM((1, CHUNK), jnp.int32),
            pltpu.VMEM((1, CHUNK), jnp.int32),
            pltpu.VMEM((1, CHUNK), jnp.float32),
            pltpu.VMEM((CHUNK, D), jnp.float32),
            pltpu.VMEM_SHARED((N, D), jnp.float32),
            pltpu.SemaphoreType.DMA,
        ],
    )


def _sc_scatter_body(support_hbm, col_hbm, row_hbm, w_hbm, out_hbm,
                     col_v, row_v, w_v, rows_v, acc, sem):
    c = lax.axis_index("c")
    s = lax.axis_index("s")
    wid = s * NC + c

    zvec = jnp.zeros((16,), jnp.float32)

    def _zrow(i, _):
        for g in range(D // 16):
            rows_v[i, pl.ds(g * 16, 16)] = zvec
        return 0

    lax.fori_loop(0, CHUNK, _zrow, 0)

    base = s * ROWS_PER_TILE

    def _zero_rows(nrows):
        nfull = nrows // CHUNK
        rem = nrows - nfull * CHUNK
        for k in range(nfull):
            pltpu.sync_copy(rows_v, acc.at[pl.ds(base + k * CHUNK, CHUNK)])
        if rem:
            pltpu.sync_copy(rows_v.at[pl.ds(0, rem)],
                            acc.at[pl.ds(base + nfull * CHUNK, rem)])

    @pl.when(s < NS - 1)
    def _():
        _zero_rows(ROWS_PER_TILE)

    @pl.when(s == NS - 1)
    def _():
        _zero_rows(ROWS_LAST)

    plsc.subcore_barrier()

    ebase = wid * CH * CHUNK

    def _chunk(j, _):
        off = ebase + j * CHUNK
        pltpu.sync_copy(col_hbm.at[pl.ds(off, CHUNK)], col_v.at[0])
        pltpu.sync_copy(row_hbm.at[pl.ds(off, CHUNK)], row_v.at[0])
        pltpu.sync_copy(w_hbm.at[pl.ds(off, CHUNK)], w_v.at[0])
        pltpu.async_copy(support_hbm.at[col_v.at[0]], rows_v, sem).wait()

        def _scale(t, _):
            wv = w_v[0, pl.ds(t * 16, 16)]
            for l in range(16):
                w = wv[l]
                e = t * 16 + l
                for g in range(D // 16):
                    sl = pl.ds(g * 16, 16)
                    rows_v[e, sl] = rows_v[e, sl] * w
            return 0

        lax.fori_loop(0, CHUNK // 16, _scale, 0)
        pltpu.sync_copy(rows_v, acc.at[row_v.at[0]], add=True)
        return 0

    lax.fori_loop(0, CH, _chunk, 0)

    plsc.subcore_barrier()

    @pl.when(s < NS - 1)
    def _():
        pltpu.sync_copy(acc.at[pl.ds(base, ROWS_PER_TILE)],
                        out_hbm.at[c, pl.ds(base, ROWS_PER_TILE)])

    @pl.when(s == NS - 1)
    def _():
        pltpu.sync_copy(acc.at[pl.ds(base, ROWS_LAST)],
                        out_hbm.at[c, pl.ds(base, ROWS_LAST)])


def kernel(input, adj_edge_index, adj_edge_weight, W, b):
    support = _matmul(input, W, b)

    row = adj_edge_index[0]
    col = adj_edge_index[1]
    pad = E_PAD - E
    col_p = jnp.pad(col, (0, pad))
    row_p = jnp.pad(row, (0, pad))
    w_p = jnp.pad(adj_edge_weight, (0, pad))

    partials = _build_sc_scatter()(support, col_p, row_p, w_p)
    return _sum_partials(partials)
